# Initial kernel scaffold; baseline (speedup 1.0000x reference)
import jax
import jax.numpy as jnp
from jax.experimental import pallas as pl


def kernel(target_word, synonym, antonym, W):
    raise NotImplementedError("write your pallas kernel here")


# speedup vs baseline: 1.0991x; 1.0991x over previous
import functools

import jax
import jax.numpy as jnp
from jax import lax
from jax.experimental import pallas as pl
from jax.experimental.pallas import tpu as pltpu
from jax.experimental.pallas import tpu_sc as plsc

NC = 2
NS = 16
NW = NC * NS

CHUNK = 128
DPAD = 384


def _gather_body(idx_hbm, out_hbm, w_hbm, idx_v, buf0, buf1, gsems, wsems,
                 wid, nchunks):
    base = pl.multiple_of(wid * nchunks * CHUNK, CHUNK)

    pltpu.sync_copy(idx_hbm.at[wid], idx_v)

    def step(i, _):
        c0 = pl.multiple_of(2 * i * CHUNK, CHUNK)
        c1 = pl.multiple_of((2 * i + 1) * CHUNK, CHUNK)
        g0 = pltpu.async_copy(w_hbm.at[idx_v.at[2 * i]], buf0, gsems[0])
        g1 = pltpu.async_copy(w_hbm.at[idx_v.at[2 * i + 1]], buf1, gsems[1])
        g0.wait()
        w0 = pltpu.async_copy(buf0, out_hbm.at[pl.ds(base + c0, CHUNK)],
                              wsems[0])
        g1.wait()
        w1 = pltpu.async_copy(buf1, out_hbm.at[pl.ds(base + c1, CHUNK)],
                              wsems[1])
        w0.wait()
        w1.wait()
        return 0

    lax.fori_loop(0, nchunks // 2, step, 0)


def _make_sc_gather(n_total, n_tensors):
    n_per_w = n_total // NW
    nchunks = n_per_w // CHUNK
    mesh = plsc.VectorSubcoreMesh(core_axis_name="c", subcore_axis_name="s")

    @functools.partial(
        pl.kernel,
        out_type=[jax.ShapeDtypeStruct((n_total, DPAD), jnp.float32)
                  for _ in range(n_tensors)],
        mesh=mesh,
        scratch_types=[
            pltpu.VMEM((nchunks, CHUNK), jnp.int32),
            pltpu.VMEM((CHUNK, DPAD), jnp.float32),
            pltpu.VMEM((CHUNK, DPAD), jnp.float32),
            pltpu.SemaphoreType.DMA,
            pltpu.SemaphoreType.DMA,
            pltpu.SemaphoreType.DMA,
            pltpu.SemaphoreType.DMA,
        ],
    )
    def sc_gather(*refs):
        idx_refs = refs[:n_tensors]
        w_hbm = refs[n_tensors]
        out_refs = refs[n_tensors + 1:2 * n_tensors + 1]
        idx_v, buf0, buf1, g0, g1, w0, w1 = refs[2 * n_tensors + 1:]
        wid = lax.axis_index("s") * NC + lax.axis_index("c")
        for idx_hbm, out_hbm in zip(idx_refs, out_refs):
            _gather_body(idx_hbm, out_hbm, w_hbm, idx_v, buf0, buf1,
                         (g0, g1), (w0, w1), wid, nchunks)

    return sc_gather


def kernel(target_word, synonym, antonym, W):
    b, l = target_word.shape
    dim = W.shape[1]
    n = b * l
    n_per_w = n // NW
    nchunks = n_per_w // CHUNK
    w_pad = jnp.pad(W, ((0, 0), (0, DPAD - dim)))

    def prep(idx):
        return idx.reshape(NW, nchunks, CHUNK).astype(jnp.int32)

    fn = _make_sc_gather(n, 3)
    outs = fn(prep(target_word), prep(synonym), prep(antonym), w_pad)
    return tuple(o[:, :dim].reshape(b, l, dim) for o in outs)
